# baseline (speedup 1.0000x reference)
import functools

import jax
import jax.numpy as jnp
from jax import lax
from jax.experimental import pallas as pl
from jax.experimental.pallas import tpu as pltpu
from jax.experimental.pallas import tpu_sc as plsc

B = 1024
CTX = 20
EMB = 64
VOC = 100000

NC = 2
NS = 16
NW = NC * NS
BPW = B // NW
ROWS = BPW * CTX
IDX_CHUNK = 128
NCHUNK = ROWS // IDX_CHUNK

TN = 2048


def _sc_pool_body(table_hbm, idx_hbm, out_hbm, idx_v, rows_v, pooled_v, sem):
    wid = lax.axis_index("s") * NC + lax.axis_index("c")

    pltpu.sync_copy(idx_hbm.at[wid], idx_v)

    copies = [
        pltpu.make_async_copy(
            table_hbm.at[idx_v.at[j]],
            rows_v.at[pl.ds(j * IDX_CHUNK, IDX_CHUNK)],
            sem,
        )
        for j in range(NCHUNK)
    ]
    for c in copies:
        c.start()
    for c in copies:
        c.wait()

    inv = jnp.float32(1.0 / CTX)

    def body(b, carry):
        base = b * CTX
        for j in range(EMB // 16):
            acc = jnp.zeros((16,), jnp.float32)
            for c in range(CTX):
                acc = acc + rows_v[base + c, pl.ds(j * 16, 16)]
            pooled_v[b, pl.ds(j * 16, 16)] = acc * inv
        return carry

    lax.fori_loop(0, BPW, body, 0)

    pltpu.sync_copy(pooled_v, out_hbm.at[pl.ds(wid * BPW, BPW)])


@functools.partial(jax.jit, static_argnames=())
def _sc_pool(emb_table, idx2d):
    kern = pl.kernel(
        _sc_pool_body,
        out_type=jax.ShapeDtypeStruct((B, EMB), jnp.float32),
        mesh=plsc.VectorSubcoreMesh(core_axis_name="c", subcore_axis_name="s"),
        scratch_types=[
            pltpu.VMEM((NCHUNK, IDX_CHUNK), jnp.int32),
            pltpu.VMEM((ROWS, EMB), jnp.float32),
            pltpu.VMEM((BPW, EMB), jnp.float32),
            pltpu.SemaphoreType.DMA,
        ],
        compiler_params=pltpu.CompilerParams(use_tc_tiling_on_sc=False),
    )
    return kern(emb_table, idx2d)


def _mm_body(p_ref, w_ref, o_ref):
    o_ref[...] = lax.dot_general(
        p_ref[...],
        w_ref[...],
        dimension_numbers=(((1,), (1,)), ((), ())),
        preferred_element_type=jnp.float32,
    )


def _tc_matmul(pooled, ffw_weight):
    grid = pl.cdiv(VOC, TN)
    return pl.pallas_call(
        _mm_body,
        grid=(grid,),
        in_specs=[
            pl.BlockSpec((B, EMB), lambda i: (0, 0)),
            pl.BlockSpec((TN, EMB), lambda i: (i, 0)),
        ],
        out_specs=pl.BlockSpec((B, TN), lambda i: (0, i)),
        out_shape=jax.ShapeDtypeStruct((B, VOC), jnp.float32),
        compiler_params=pltpu.CompilerParams(
            dimension_semantics=("arbitrary",),
        ),
    )(pooled, ffw_weight)


def kernel(inpt, emb_table, ffw_weight):
    idx = inpt.astype(jnp.int32).reshape(NW, NCHUNK, IDX_CHUNK)
    pooled = _sc_pool(emb_table, idx)
    return _tc_matmul(pooled, ffw_weight)


# speedup vs baseline: 1.0051x; 1.0051x over previous
import functools

import jax
import jax.numpy as jnp
from jax import lax
from jax.experimental import pallas as pl
from jax.experimental.pallas import tpu as pltpu
from jax.experimental.pallas import tpu_sc as plsc

B = 1024
CTX = 20
EMB = 64
VOC = 100000

NC = 2
NS = 16
NW = NC * NS
BPW = B // NW
ROWS = BPW * CTX
IDX_CHUNK = 128
NCHUNK = ROWS // IDX_CHUNK

TN = 4096


def _sc_pool_body(table_hbm, idx_hbm, out_hbm, idx_v, rows_v, pooled_v, sem):
    wid = lax.axis_index("s") * NC + lax.axis_index("c")

    pltpu.sync_copy(idx_hbm.at[wid], idx_v)

    copies = [
        pltpu.make_async_copy(
            table_hbm.at[idx_v.at[j]],
            rows_v.at[pl.ds(j * IDX_CHUNK, IDX_CHUNK)],
            sem,
        )
        for j in range(NCHUNK)
    ]
    for c in copies:
        c.start()
    for c in copies:
        c.wait()

    inv = jnp.float32(1.0 / CTX)

    def body(b, carry):
        base = b * CTX
        for j in range(EMB // 16):
            acc = jnp.zeros((16,), jnp.float32)
            for c in range(CTX):
                acc = acc + rows_v[base + c, pl.ds(j * 16, 16)]
            pooled_v[b, pl.ds(j * 16, 16)] = acc * inv
        return carry

    lax.fori_loop(0, BPW, body, 0)

    pltpu.sync_copy(pooled_v, out_hbm.at[pl.ds(wid * BPW, BPW)])


@functools.partial(jax.jit, static_argnames=())
def _sc_pool(emb_table, idx2d):
    kern = pl.kernel(
        _sc_pool_body,
        out_type=jax.ShapeDtypeStruct((B, EMB), jnp.float32),
        mesh=plsc.VectorSubcoreMesh(core_axis_name="c", subcore_axis_name="s"),
        scratch_types=[
            pltpu.VMEM((NCHUNK, IDX_CHUNK), jnp.int32),
            pltpu.VMEM((ROWS, EMB), jnp.float32),
            pltpu.VMEM((BPW, EMB), jnp.float32),
            pltpu.SemaphoreType.DMA,
        ],
        compiler_params=pltpu.CompilerParams(use_tc_tiling_on_sc=False),
    )
    return kern(emb_table, idx2d)


def _mm_body(p_ref, w_ref, o_ref):
    o_ref[...] = lax.dot_general(
        p_ref[...],
        w_ref[...],
        dimension_numbers=(((1,), (1,)), ((), ())),
        preferred_element_type=jnp.float32,
    )


def _tc_matmul(pooled, ffw_weight):
    grid = pl.cdiv(VOC, TN)
    return pl.pallas_call(
        _mm_body,
        grid=(grid,),
        in_specs=[
            pl.BlockSpec((B, EMB), lambda i: (0, 0)),
            pl.BlockSpec((TN, EMB), lambda i: (i, 0)),
        ],
        out_specs=pl.BlockSpec((B, TN), lambda i: (0, i)),
        out_shape=jax.ShapeDtypeStruct((B, VOC), jnp.float32),
        compiler_params=pltpu.CompilerParams(
            dimension_semantics=("arbitrary",),
        ),
    )(pooled, ffw_weight)


def kernel(inpt, emb_table, ffw_weight):
    idx = inpt.astype(jnp.int32).reshape(NW, NCHUNK, IDX_CHUNK)
    pooled = _sc_pool(emb_table, idx)
    return _tc_matmul(pooled, ffw_weight)
